# baseline (speedup 1.0000x reference)
import functools

import jax
import jax.numpy as jnp
from jax import lax
from jax.experimental import pallas as pl
from jax.experimental.pallas import tpu as pltpu
from jax.experimental.pallas import tpu_sc as plsc

_BR = 1024


def kernel(x):
    n, d = x.shape
    info = plsc.get_sparse_core_info()
    nc, ns = info.num_cores, info.num_subcores
    nw = nc * ns
    rows_w = n // nw
    mesh = plsc.VectorSubcoreMesh(core_axis_name="c", subcore_axis_name="s")

    @functools.partial(
        pl.kernel,
        mesh=mesh,
        out_type=jax.ShapeDtypeStruct((n, d), x.dtype),
        scratch_types=[
            pltpu.VMEM((_BR, d), x.dtype),
        ],
    )
    def _copy(x_hbm, o_hbm, buf):
        wid = lax.axis_index("s") * nc + lax.axis_index("c")
        base = wid * rows_w
        for j in range(rows_w // _BR):
            pltpu.sync_copy(x_hbm.at[pl.ds(base + j * _BR, _BR), :], buf)
            pltpu.sync_copy(buf, o_hbm.at[pl.ds(base + j * _BR, _BR), :])

    return _copy(x)
